# Initial kernel scaffold; baseline (speedup 1.0000x reference)
import functools

import jax
import jax.numpy as jnp
from jax import lax
from jax.experimental import pallas as pl
from jax.experimental.pallas import tpu as pltpu
from jax.experimental.pallas import tpu_sc as plsc

_NC = 2
_NS = 16
_NW = _NC * _NS
_ECH = 128
_H = 64
_W1A = 80
_W2A = 16
_MM_TILE = 400
_ROW_TILE = 400


def _mm1_body(x_ref, wt_ref, xlaug_ref, xr_ref):
    y = jnp.dot(x_ref[...], wt_ref[...], preferred_element_type=jnp.float32)
    tile = y.shape[0]
    xlaug_ref[:, :_H] = y[:, :_H]
    col = lax.broadcasted_iota(jnp.int32, (tile, _W1A - _H), 1)
    xlaug_ref[:, _H:] = jnp.where(col == 0, 1.0, 0.0)
    xr_ref[...] = y[:, _H:]


def _mid_body(p_ref, xr_ref, b1_ref, w2lt_ref, w2rt_ref, hl_ref, hrc_ref):
    agg = p_ref[0] + p_ref[1]
    cnt = jnp.maximum(agg[:, _H:_H + 1], 1.0)
    h = jnp.maximum(agg[:, :_H] / cnt + b1_ref[...] + xr_ref[...], 0.0)
    hl_ref[...] = jnp.dot(h, w2lt_ref[...], preferred_element_type=jnp.float32)
    hr = jnp.dot(h, w2rt_ref[...], preferred_element_type=jnp.float32)
    col = lax.broadcasted_iota(jnp.int32, hr.shape, 1)
    hrc_ref[...] = jnp.where(col == 7, jnp.broadcast_to(cnt, hr.shape), hr)


def _out_body(q_ref, hrc_ref, b2_ref, out_ref):
    agg = q_ref[0] + q_ref[1]
    cnt = hrc_ref[:, 7:8]
    logits = agg[:, :7] / cnt + b2_ref[...] + hrc_ref[:, :7]
    m = jnp.max(logits, axis=1, keepdims=True)
    s = logits - m
    lse = jnp.log(jnp.sum(jnp.exp(s), axis=1, keepdims=True))
    out_ref[...] = s - lse


def _sc_segment_sum(feat, srcr, dstr, zeros, chunks):
    n_pad, d = zeros.shape
    rows_pt = n_pad // _NS
    mesh = plsc.VectorSubcoreMesh(core_axis_name="c", subcore_axis_name="s")

    @functools.partial(
        pl.kernel,
        out_type=jax.ShapeDtypeStruct((_NC, n_pad, d), jnp.float32),
        mesh=mesh,
        scratch_types=[
            pltpu.VMEM((chunks, _ECH), jnp.int32),
            pltpu.VMEM((chunks, _ECH), jnp.int32),
            pltpu.VMEM((_ECH, d), jnp.float32),
            pltpu.VMEM_SHARED((n_pad, d), jnp.float32),
        ],
    )
    def kern(feat_hbm, src_hbm, dst_hbm, z_hbm, out_hbm, src_v, dst_v, buf, acc):
        c = lax.axis_index("c")
        s = lax.axis_index("s")
        wid = c * _NS + s
        pltpu.sync_copy(z_hbm.at[pl.ds(s * rows_pt, rows_pt)],
                        acc.at[pl.ds(s * rows_pt, rows_pt)])
        pltpu.sync_copy(src_hbm.at[wid], src_v)
        pltpu.sync_copy(dst_hbm.at[wid], dst_v)
        plsc.subcore_barrier()

        @pl.loop(0, chunks)
        def _(j):
            pltpu.sync_copy(feat_hbm.at[src_v.at[j]], buf)
            pltpu.sync_copy(buf, acc.at[dst_v.at[j]], add=True)

        plsc.subcore_barrier()
        pltpu.sync_copy(acc.at[pl.ds(s * rows_pt, rows_pt)],
                        out_hbm.at[c, pl.ds(s * rows_pt, rows_pt)])

    return kern(feat, srcr, dstr, zeros)


def kernel(x, edge_index, W1l, b1, W1r, W2l, b2, W2r):
    n, f_in = x.shape
    e = edge_index.shape[1]
    n_pad = ((n + 1 + _NS * 8 - 1) // (_NS * 8)) * (_NS * 8)
    per = _NW * _ECH
    e_pad = ((e + per - 1) // per) * per
    chunks = e_pad // per

    src = jnp.concatenate([edge_index[0], jnp.zeros((e_pad - e,), jnp.int32)])
    dst = jnp.concatenate([edge_index[1], jnp.full((e_pad - e,), n, jnp.int32)])
    srcr = src.reshape(_NW, chunks, _ECH)
    dstr = dst.reshape(_NW, chunks, _ECH)

    w1t = jnp.concatenate([W1l, W1r], axis=0).T
    w2lt = jnp.pad(W2l, ((0, _W2A - W2l.shape[0]), (0, 0))).T
    w2rt = jnp.pad(W2r, ((0, 1), (0, 0))).T
    b1r = b1.reshape(1, _H)
    b2r = b2.reshape(1, 7)
    zeros1 = jnp.zeros((n_pad, _W1A), jnp.float32)
    zeros2 = jnp.zeros((n_pad, _W2A), jnp.float32)

    grid_mm = n // _MM_TILE
    xlaug, xr = pl.pallas_call(
        _mm1_body,
        grid=(grid_mm,),
        in_specs=[
            pl.BlockSpec((_MM_TILE, f_in), lambda i: (i, 0)),
            pl.BlockSpec((f_in, 2 * _H), lambda i: (0, 0)),
        ],
        out_specs=[
            pl.BlockSpec((_MM_TILE, _W1A), lambda i: (i, 0)),
            pl.BlockSpec((_MM_TILE, _H), lambda i: (i, 0)),
        ],
        out_shape=[
            jax.ShapeDtypeStruct((n, _W1A), jnp.float32),
            jax.ShapeDtypeStruct((n, _H), jnp.float32),
        ],
    )(x, w1t)

    parts1 = _sc_segment_sum(xlaug, srcr, dstr, zeros1, chunks)

    grid_r = n // _ROW_TILE
    hl, hrc = pl.pallas_call(
        _mid_body,
        grid=(grid_r,),
        in_specs=[
            pl.BlockSpec((_NC, _ROW_TILE, _W1A), lambda i: (0, i, 0)),
            pl.BlockSpec((_ROW_TILE, _H), lambda i: (i, 0)),
            pl.BlockSpec((1, _H), lambda i: (0, 0)),
            pl.BlockSpec((_H, _W2A), lambda i: (0, 0)),
            pl.BlockSpec((_H, 8), lambda i: (0, 0)),
        ],
        out_specs=[
            pl.BlockSpec((_ROW_TILE, _W2A), lambda i: (i, 0)),
            pl.BlockSpec((_ROW_TILE, 8), lambda i: (i, 0)),
        ],
        out_shape=[
            jax.ShapeDtypeStruct((n, _W2A), jnp.float32),
            jax.ShapeDtypeStruct((n, 8), jnp.float32),
        ],
    )(parts1, xr, b1r, w2lt, w2rt)

    parts2 = _sc_segment_sum(hl, srcr, dstr, zeros2, chunks)

    out = pl.pallas_call(
        _out_body,
        grid=(grid_r,),
        in_specs=[
            pl.BlockSpec((_NC, _ROW_TILE, _W2A), lambda i: (0, i, 0)),
            pl.BlockSpec((_ROW_TILE, 8), lambda i: (i, 0)),
            pl.BlockSpec((1, 7), lambda i: (0, 0)),
        ],
        out_specs=pl.BlockSpec((_ROW_TILE, 7), lambda i: (i, 0)),
        out_shape=jax.ShapeDtypeStruct((n, 7), jnp.float32),
    )(parts2, hrc, b2r)
    return out


# speedup vs baseline: 7.9228x; 7.9228x over previous
import functools

import jax
import jax.numpy as jnp
from jax import lax
from jax.experimental import pallas as pl
from jax.experimental.pallas import tpu as pltpu
from jax.experimental.pallas import tpu_sc as plsc

_NC = 2
_NS = 16
_NW = _NC * _NS
_ECH = 128
_H = 64
_W1A = 80
_W2A = 16
_MM_TILE = 400
_ROW_TILE = 400


def _mm1_body(x_ref, wt_ref, xlaug_ref, xr_ref):
    y = jnp.dot(x_ref[...], wt_ref[...], preferred_element_type=jnp.float32)
    tile = y.shape[0]
    xlaug_ref[:, :_H] = y[:, :_H]
    col = lax.broadcasted_iota(jnp.int32, (tile, _W1A - _H), 1)
    xlaug_ref[:, _H:] = jnp.where(col == 0, 1.0, 0.0)
    xr_ref[...] = y[:, _H:]


def _mid_body(p_ref, xr_ref, b1_ref, w2lt_ref, w2rt_ref, hl_ref, hrc_ref):
    agg = p_ref[0] + p_ref[1]
    cnt = jnp.maximum(agg[:, _H:_H + 1], 1.0)
    h = jnp.maximum(agg[:, :_H] / cnt + b1_ref[...] + xr_ref[...], 0.0)
    hl_ref[...] = jnp.dot(h, w2lt_ref[...], preferred_element_type=jnp.float32)
    hr = jnp.dot(h, w2rt_ref[...], preferred_element_type=jnp.float32)
    col = lax.broadcasted_iota(jnp.int32, hr.shape, 1)
    hrc_ref[...] = jnp.where(col == 7, jnp.broadcast_to(cnt, hr.shape), hr)


def _out_body(q_ref, hrc_ref, b2_ref, out_ref):
    agg = q_ref[0] + q_ref[1]
    cnt = hrc_ref[:, 7:8]
    logits = agg[:, :7] / cnt + b2_ref[...] + hrc_ref[:, :7]
    m = jnp.max(logits, axis=1, keepdims=True)
    s = logits - m
    lse = jnp.log(jnp.sum(jnp.exp(s), axis=1, keepdims=True))
    out_ref[...] = s - lse


def _sc_segment_sum(feat, srcr, dstr, zeros, chunks):
    n_pad, d = zeros.shape
    rows_pt = n_pad // _NS
    mesh = plsc.VectorSubcoreMesh(core_axis_name="c", subcore_axis_name="s")

    @functools.partial(
        pl.kernel,
        out_type=jax.ShapeDtypeStruct((_NC, n_pad, d), jnp.float32),
        mesh=mesh,
        compiler_params=pltpu.CompilerParams(use_tc_tiling_on_sc=False),
        scratch_types=[
            pltpu.VMEM((chunks, _ECH), jnp.int32),
            pltpu.VMEM((chunks, _ECH), jnp.int32),
            pltpu.VMEM((_ECH, d), jnp.float32),
            pltpu.VMEM_SHARED((n_pad, d), jnp.float32),
        ],
    )
    def kern(feat_hbm, src_hbm, dst_hbm, z_hbm, out_hbm, src_v, dst_v, buf, acc):
        c = lax.axis_index("c")
        s = lax.axis_index("s")
        wid = c * _NS + s
        pltpu.sync_copy(z_hbm.at[pl.ds(s * rows_pt, rows_pt)],
                        acc.at[pl.ds(s * rows_pt, rows_pt)])
        pltpu.sync_copy(src_hbm.at[wid], src_v)
        pltpu.sync_copy(dst_hbm.at[wid], dst_v)
        plsc.subcore_barrier()

        @pl.loop(0, chunks)
        def _(j):
            pltpu.sync_copy(feat_hbm.at[src_v.at[j]], buf)
            pltpu.sync_copy(buf, acc.at[dst_v.at[j]], add=True)

        plsc.subcore_barrier()
        pltpu.sync_copy(acc.at[pl.ds(s * rows_pt, rows_pt)],
                        out_hbm.at[c, pl.ds(s * rows_pt, rows_pt)])

    return kern(feat, srcr, dstr, zeros)


def kernel(x, edge_index, W1l, b1, W1r, W2l, b2, W2r):
    n, f_in = x.shape
    e = edge_index.shape[1]
    n_pad = ((n + 1 + _NS * 8 - 1) // (_NS * 8)) * (_NS * 8)
    per = _NW * _ECH
    e_pad = ((e + per - 1) // per) * per
    chunks = e_pad // per

    src = jnp.concatenate([edge_index[0], jnp.zeros((e_pad - e,), jnp.int32)])
    dst = jnp.concatenate([edge_index[1], jnp.full((e_pad - e,), n, jnp.int32)])
    srcr = src.reshape(_NW, chunks, _ECH)
    dstr = dst.reshape(_NW, chunks, _ECH)

    w1t = jnp.concatenate([W1l, W1r], axis=0).T
    w2lt = jnp.pad(W2l, ((0, _W2A - W2l.shape[0]), (0, 0))).T
    w2rt = jnp.pad(W2r, ((0, 1), (0, 0))).T
    b1r = b1.reshape(1, _H)
    b2r = b2.reshape(1, 7)
    zeros1 = jnp.zeros((n_pad, _W1A), jnp.float32)
    zeros2 = jnp.zeros((n_pad, _W2A), jnp.float32)

    grid_mm = n // _MM_TILE
    xlaug, xr = pl.pallas_call(
        _mm1_body,
        grid=(grid_mm,),
        in_specs=[
            pl.BlockSpec((_MM_TILE, f_in), lambda i: (i, 0)),
            pl.BlockSpec((f_in, 2 * _H), lambda i: (0, 0)),
        ],
        out_specs=[
            pl.BlockSpec((_MM_TILE, _W1A), lambda i: (i, 0)),
            pl.BlockSpec((_MM_TILE, _H), lambda i: (i, 0)),
        ],
        out_shape=[
            jax.ShapeDtypeStruct((n, _W1A), jnp.float32),
            jax.ShapeDtypeStruct((n, _H), jnp.float32),
        ],
    )(x, w1t)

    parts1 = _sc_segment_sum(xlaug, srcr, dstr, zeros1, chunks)

    grid_r = n // _ROW_TILE
    hl, hrc = pl.pallas_call(
        _mid_body,
        grid=(grid_r,),
        in_specs=[
            pl.BlockSpec((_NC, _ROW_TILE, _W1A), lambda i: (0, i, 0)),
            pl.BlockSpec((_ROW_TILE, _H), lambda i: (i, 0)),
            pl.BlockSpec((1, _H), lambda i: (0, 0)),
            pl.BlockSpec((_H, _W2A), lambda i: (0, 0)),
            pl.BlockSpec((_H, 8), lambda i: (0, 0)),
        ],
        out_specs=[
            pl.BlockSpec((_ROW_TILE, _W2A), lambda i: (i, 0)),
            pl.BlockSpec((_ROW_TILE, 8), lambda i: (i, 0)),
        ],
        out_shape=[
            jax.ShapeDtypeStruct((n, _W2A), jnp.float32),
            jax.ShapeDtypeStruct((n, 8), jnp.float32),
        ],
    )(parts1, xr, b1r, w2lt, w2rt)

    parts2 = _sc_segment_sum(hl, srcr, dstr, zeros2, chunks)

    out = pl.pallas_call(
        _out_body,
        grid=(grid_r,),
        in_specs=[
            pl.BlockSpec((_NC, _ROW_TILE, _W2A), lambda i: (0, i, 0)),
            pl.BlockSpec((_ROW_TILE, 8), lambda i: (i, 0)),
            pl.BlockSpec((1, 7), lambda i: (0, 0)),
        ],
        out_specs=pl.BlockSpec((_ROW_TILE, 7), lambda i: (i, 0)),
        out_shape=jax.ShapeDtypeStruct((n, 7), jnp.float32),
    )(parts2, hrc, b2r)
    return out


# speedup vs baseline: 8.3736x; 1.0569x over previous
import functools

import jax
import jax.numpy as jnp
from jax import lax
from jax.experimental import pallas as pl
from jax.experimental.pallas import tpu as pltpu
from jax.experimental.pallas import tpu_sc as plsc

_NC = 2
_NS = 16
_NW = _NC * _NS
_ECH = 128
_H = 64
_W1A = 80
_W2A = 16
_MM_TILE = 400
_ROW_TILE = 400


def _mm1_body(x_ref, wt_ref, xlaug_ref, xr_ref):
    y = jnp.dot(x_ref[...], wt_ref[...], preferred_element_type=jnp.float32)
    tile = y.shape[0]
    xlaug_ref[:, :_H] = y[:, :_H]
    col = lax.broadcasted_iota(jnp.int32, (tile, _W1A - _H), 1)
    xlaug_ref[:, _H:] = jnp.where(col == 0, 1.0, 0.0)
    xr_ref[...] = y[:, _H:]


def _mid_body(p_ref, xr_ref, b1_ref, w2lt_ref, w2rt_ref, hl_ref, hrc_ref):
    agg = p_ref[0] + p_ref[1]
    cnt = jnp.maximum(agg[:, _H:_H + 1], 1.0)
    h = jnp.maximum(agg[:, :_H] / cnt + b1_ref[...] + xr_ref[...], 0.0)
    hl_ref[...] = jnp.dot(h, w2lt_ref[...], preferred_element_type=jnp.float32)
    hr = jnp.dot(h, w2rt_ref[...], preferred_element_type=jnp.float32)
    col = lax.broadcasted_iota(jnp.int32, hr.shape, 1)
    hrc_ref[...] = jnp.where(col == 7, jnp.broadcast_to(cnt, hr.shape), hr)


def _out_body(q_ref, hrc_ref, b2_ref, out_ref):
    agg = q_ref[0] + q_ref[1]
    cnt = hrc_ref[:, 7:8]
    logits = agg[:, :7] / cnt + b2_ref[...] + hrc_ref[:, :7]
    m = jnp.max(logits, axis=1, keepdims=True)
    s = logits - m
    lse = jnp.log(jnp.sum(jnp.exp(s), axis=1, keepdims=True))
    out_ref[...] = s - lse


def _sc_segment_sum(feat, srcr, dstr, zeros, chunks):
    n_pad, d = zeros.shape
    rows_pt = n_pad // _NS
    budget = 2_000_000 - n_pad * d - _NS * 2 * chunks * _ECH
    g = max(1, min(7, chunks, budget // (_NS * 2 * _ECH * d)))
    groups = [(lo, min(lo + g, chunks)) for lo in range(0, chunks, g)]
    mesh = plsc.VectorSubcoreMesh(core_axis_name="c", subcore_axis_name="s")

    @functools.partial(
        pl.kernel,
        out_type=jax.ShapeDtypeStruct((_NC, n_pad, d), jnp.float32),
        mesh=mesh,
        compiler_params=pltpu.CompilerParams(use_tc_tiling_on_sc=False),
        scratch_types=[
            pltpu.VMEM((chunks, _ECH), jnp.int32),
            pltpu.VMEM((chunks, _ECH), jnp.int32),
            pltpu.VMEM((2, g * _ECH, d), jnp.float32),
            pltpu.VMEM_SHARED((n_pad, d), jnp.float32),
            pltpu.SemaphoreType.DMA,
            pltpu.SemaphoreType.DMA,
            pltpu.SemaphoreType.DMA,
        ],
    )
    def kern(feat_hbm, src_hbm, dst_hbm, z_hbm, out_hbm,
             src_v, dst_v, bufs, acc, semz, semg, sems):
        c = lax.axis_index("c")
        s = lax.axis_index("s")
        wid = c * _NS + s
        rows = pl.ds(s * rows_pt, rows_pt)
        zh = pltpu.async_copy(z_hbm.at[rows], acc.at[rows], semz)
        pltpu.sync_copy(src_hbm.at[wid], src_v)
        pltpu.sync_copy(dst_hbm.at[wid], dst_v)

        def gather(j, lo, buf_i):
            return pltpu.async_copy(
                feat_hbm.at[src_v.at[j]],
                bufs.at[buf_i].at[pl.ds((j - lo) * _ECH, _ECH)], semg)

        def scatter_add(j, lo, buf_i):
            return pltpu.async_copy(
                bufs.at[buf_i].at[pl.ds((j - lo) * _ECH, _ECH)],
                acc.at[dst_v.at[j]], sems, add=True)

        gh, sh = {}, {}
        lo0, hi0 = groups[0]
        for j in range(lo0, hi0):
            gh[j] = gather(j, lo0, 0)
        zh.wait()
        plsc.subcore_barrier()

        for gi, (lo, hi) in enumerate(groups):
            for j in range(lo, hi):
                gh[j].wait()
            if gi >= 1:
                plo, phi = groups[gi - 1]
                for j in range(plo, phi):
                    sh[j].wait()
            if gi + 1 < len(groups):
                nlo, nhi = groups[gi + 1]
                for j in range(nlo, nhi):
                    gh[j] = gather(j, nlo, (gi + 1) % 2)
            for j in range(lo, hi):
                sh[j] = scatter_add(j, lo, gi % 2)
        for j in range(groups[-1][0], groups[-1][1]):
            sh[j].wait()
        plsc.subcore_barrier()
        pltpu.sync_copy(acc.at[rows], out_hbm.at[c, rows])

    return kern(feat, srcr, dstr, zeros)


def kernel(x, edge_index, W1l, b1, W1r, W2l, b2, W2r):
    n, f_in = x.shape
    e = edge_index.shape[1]
    n_pad = ((n + 1 + _NS * 8 - 1) // (_NS * 8)) * (_NS * 8)
    per = _NW * _ECH
    e_pad = ((e + per - 1) // per) * per
    chunks = e_pad // per

    src = jnp.concatenate([edge_index[0], jnp.zeros((e_pad - e,), jnp.int32)])
    dst = jnp.concatenate([edge_index[1], jnp.full((e_pad - e,), n, jnp.int32)])
    srcr = src.reshape(_NW, chunks, _ECH)
    dstr = dst.reshape(_NW, chunks, _ECH)

    w1t = jnp.concatenate([W1l, W1r], axis=0).T
    w2lt = jnp.pad(W2l, ((0, _W2A - W2l.shape[0]), (0, 0))).T
    w2rt = jnp.pad(W2r, ((0, 1), (0, 0))).T
    b1r = b1.reshape(1, _H)
    b2r = b2.reshape(1, 7)
    zeros1 = jnp.zeros((n_pad, _W1A), jnp.float32)
    zeros2 = jnp.zeros((n_pad, _W2A), jnp.float32)

    grid_mm = n // _MM_TILE
    xlaug, xr = pl.pallas_call(
        _mm1_body,
        grid=(grid_mm,),
        in_specs=[
            pl.BlockSpec((_MM_TILE, f_in), lambda i: (i, 0)),
            pl.BlockSpec((f_in, 2 * _H), lambda i: (0, 0)),
        ],
        out_specs=[
            pl.BlockSpec((_MM_TILE, _W1A), lambda i: (i, 0)),
            pl.BlockSpec((_MM_TILE, _H), lambda i: (i, 0)),
        ],
        out_shape=[
            jax.ShapeDtypeStruct((n, _W1A), jnp.float32),
            jax.ShapeDtypeStruct((n, _H), jnp.float32),
        ],
    )(x, w1t)

    parts1 = _sc_segment_sum(xlaug, srcr, dstr, zeros1, chunks)

    grid_r = n // _ROW_TILE
    hl, hrc = pl.pallas_call(
        _mid_body,
        grid=(grid_r,),
        in_specs=[
            pl.BlockSpec((_NC, _ROW_TILE, _W1A), lambda i: (0, i, 0)),
            pl.BlockSpec((_ROW_TILE, _H), lambda i: (i, 0)),
            pl.BlockSpec((1, _H), lambda i: (0, 0)),
            pl.BlockSpec((_H, _W2A), lambda i: (0, 0)),
            pl.BlockSpec((_H, 8), lambda i: (0, 0)),
        ],
        out_specs=[
            pl.BlockSpec((_ROW_TILE, _W2A), lambda i: (i, 0)),
            pl.BlockSpec((_ROW_TILE, 8), lambda i: (i, 0)),
        ],
        out_shape=[
            jax.ShapeDtypeStruct((n, _W2A), jnp.float32),
            jax.ShapeDtypeStruct((n, 8), jnp.float32),
        ],
    )(parts1, xr, b1r, w2lt, w2rt)

    parts2 = _sc_segment_sum(hl, srcr, dstr, zeros2, chunks)

    out = pl.pallas_call(
        _out_body,
        grid=(grid_r,),
        in_specs=[
            pl.BlockSpec((_NC, _ROW_TILE, _W2A), lambda i: (0, i, 0)),
            pl.BlockSpec((_ROW_TILE, 8), lambda i: (i, 0)),
            pl.BlockSpec((1, 7), lambda i: (0, 0)),
        ],
        out_specs=pl.BlockSpec((_ROW_TILE, 7), lambda i: (i, 0)),
        out_shape=jax.ShapeDtypeStruct((n, 7), jnp.float32),
    )(parts2, hrc, b2r)
    return out
